# baseline (speedup 1.0000x reference)
import functools

import jax
import jax.numpy as jnp
from jax import lax
from jax.experimental import pallas as pl
from jax.experimental.pallas import tpu as pltpu
from jax.experimental.pallas import tpu_sc as plsc

_NSTREAM = 4
_UNROLL = 512
_L = 16
_SC_UNROLL = 4
_TC_BATCHES = 12


def _nn_body(pred_ref, predI_ref, tgt_ref, outI_ref, outidx_ref):
    T = tgt_ref[0]
    S, LN = T.shape
    n_pred = pred_ref.shape[2]
    per_iter = _NSTREAM * _UNROLL

    def step(k, carry):
        rmin, ridx, rI = carry
        rmin, ridx, rI = list(rmin), list(ridx), list(rI)
        for u in range(_UNROLL):
            for r in range(_NSTREAM):
                i = k * per_iter + u * _NSTREAM + r
                p = pred_ref[0, 0, i]
                iv = predI_ref[0, 0, i]
                d = jnp.abs(T - p)
                upd = d < rmin[r]
                rmin[r] = jnp.minimum(d, rmin[r])
                ridx[r] = jnp.where(upd, i, ridx[r])
                rI[r] = jnp.where(upd, iv, rI[r])
        return tuple(rmin), tuple(ridx), tuple(rI)

    init = (
        tuple(jnp.full((S, LN), jnp.inf, jnp.float32) for _ in range(_NSTREAM)),
        tuple(jnp.zeros((S, LN), jnp.int32) for _ in range(_NSTREAM)),
        tuple(jnp.zeros((S, LN), jnp.float32) for _ in range(_NSTREAM)),
    )
    rmin, ridx, rI = lax.fori_loop(0, n_pred // per_iter, step, init)

    bd, bi, bI = rmin[0], ridx[0], rI[0]
    for r in range(1, _NSTREAM):
        better = (rmin[r] < bd) | ((rmin[r] == bd) & (ridx[r] < bi))
        bd = jnp.where(better, rmin[r], bd)
        bi = jnp.where(better, ridx[r], bi)
        bI = jnp.where(better, rI[r], bI)
    outI_ref[0] = bI
    outidx_ref[0] = bi


def _tc_nn(pred_mz, pred_I, tgt_mz, nb):
    B, Np = pred_mz.shape
    _, Nt = tgt_mz.shape
    S = 8
    LN = Nt // S
    tgt3 = tgt_mz.reshape(B, S, LN)
    pred3 = pred_mz.reshape(B, 1, Np)
    predI3 = pred_I.reshape(B, 1, Np)
    matched_I3, matched_idx3 = pl.pallas_call(
        _nn_body,
        grid=(nb,),
        in_specs=[
            pl.BlockSpec((1, 1, Np), lambda b: (b, 0, 0), memory_space=pltpu.SMEM),
            pl.BlockSpec((1, 1, Np), lambda b: (b, 0, 0), memory_space=pltpu.SMEM),
            pl.BlockSpec((1, S, LN), lambda b: (b, 0, 0)),
        ],
        out_specs=(
            pl.BlockSpec((1, S, LN), lambda b: (b, 0, 0)),
            pl.BlockSpec((1, S, LN), lambda b: (b, 0, 0)),
        ),
        out_shape=(
            jax.ShapeDtypeStruct((nb, S, LN), jnp.float32),
            jax.ShapeDtypeStruct((nb, S, LN), jnp.int32),
        ),
    )(pred3, predI3, tgt3)
    return matched_I3.reshape(nb, Nt), matched_idx3.reshape(nb, Nt)


def _sc_nn(pred_mz, pred_I, tgt_mz, b0):
    B, Nt = tgt_mz.shape
    _, Np = pred_mz.shape
    Bs = B - b0
    info = plsc.get_sparse_core_info()
    nc, ns = info.num_cores, info.num_subcores
    nw = nc * ns
    chunk = (Bs * Nt) // nw
    per_b = Nt // chunk
    n_tv = chunk // _L
    mesh = plsc.VectorSubcoreMesh(core_axis_name="c", subcore_axis_name="s")

    @functools.partial(
        pl.kernel,
        mesh=mesh,
        out_type=(
            jax.ShapeDtypeStruct((Bs, Nt), jnp.float32),
            jax.ShapeDtypeStruct((Bs, Nt), jnp.int32),
        ),
        scratch_types=[
            pltpu.VMEM((Np,), jnp.float32),
            pltpu.VMEM((Np,), jnp.float32),
            pltpu.VMEM((chunk,), jnp.float32),
            pltpu.VMEM((chunk,), jnp.float32),
            pltpu.VMEM((chunk,), jnp.int32),
        ],
    )
    def body(pred_hbm, predI_hbm, tgt_hbm, outI_hbm, outidx_hbm,
             tab_v, tabI_v, tgt_v, oI_v, oidx_v):
        wid = lax.axis_index("s") * nc + lax.axis_index("c")
        bo = wid // per_b
        b = bo + b0
        h = wid % per_b
        pltpu.sync_copy(pred_hbm.at[b], tab_v)
        pltpu.sync_copy(predI_hbm.at[b], tabI_v)
        pltpu.sync_copy(tgt_hbm.at[b, pl.ds(h * chunk, chunk)], tgt_v)

        for g in range(n_tv // 8):
            Ts = [tgt_v[pl.ds((g * 8 + q) * _L, _L)] for q in range(8)]

            def step(k, carry):
                dmin, idx, ival = carry
                dmin, idx, ival = list(dmin), list(idx), list(ival)
                base = k * _L
                p16 = tab_v[pl.ds(base, _L)]
                i16 = tabI_v[pl.ds(base, _L)]
                for u in range(_L):
                    p = p16[u]
                    pI = i16[u]
                    i = base + u
                    for q in range(8):
                        d = jnp.abs(Ts[q] - p)
                        upd = d < dmin[q]
                        dmin[q] = jnp.minimum(d, dmin[q])
                        idx[q] = jnp.where(upd, i, idx[q])
                        ival[q] = jnp.where(upd, pI, ival[q])
                return tuple(dmin), tuple(idx), tuple(ival)

            init = (
                tuple(jnp.full((_L,), jnp.inf, jnp.float32) for _ in range(8)),
                tuple(jnp.zeros((_L,), jnp.int32) for _ in range(8)),
                tuple(jnp.zeros((_L,), jnp.float32) for _ in range(8)),
            )
            dmin, idx, ival = lax.fori_loop(0, Np // _L, step, init)
            for q in range(8):
                oidx_v[pl.ds((g * 8 + q) * _L, _L)] = idx[q]
                oI_v[pl.ds((g * 8 + q) * _L, _L)] = ival[q]

        pltpu.sync_copy(oI_v, outI_hbm.at[bo, pl.ds(h * chunk, chunk)])
        pltpu.sync_copy(oidx_v, outidx_hbm.at[bo, pl.ds(h * chunk, chunk)])

    return body(pred_mz, pred_I, tgt_mz)


@jax.jit
def kernel(pred_mz, pred_I, tgt_mz):
    bt = _TC_BATCHES
    sc_I, sc_idx = _sc_nn(pred_mz, pred_I, tgt_mz, bt)
    tc_I, tc_idx = _tc_nn(pred_mz, pred_I, tgt_mz, bt)
    matched_I = jnp.concatenate([tc_I, sc_I], axis=0)
    matched_idx = jnp.concatenate([tc_idx, sc_idx], axis=0)
    return matched_I, matched_idx


# speedup vs baseline: 1.3818x; 1.3818x over previous
import functools

import jax
import jax.numpy as jnp
from jax import lax
from jax.experimental import pallas as pl
from jax.experimental.pallas import tpu as pltpu
from jax.experimental.pallas import tpu_sc as plsc

_NSTREAM = 4
_UNROLL = 256
_L = 16
_SC_UNROLL = 4
_TC_BATCHES = 12


def _nn_body(pred_ref, predI_ref, tgt_ref, outI_ref, outidx_ref):
    T = tgt_ref[0]
    S, LN = T.shape
    n_pred = pred_ref.shape[2]
    per_iter = _NSTREAM * _UNROLL

    def step(k, carry):
        rmin, ridx, rI = carry
        rmin, ridx, rI = list(rmin), list(ridx), list(rI)
        for u in range(_UNROLL):
            for r in range(_NSTREAM):
                i = k * per_iter + u * _NSTREAM + r
                p = pred_ref[0, 0, i]
                iv = predI_ref[0, 0, i]
                d = jnp.abs(T - p)
                upd = d < rmin[r]
                rmin[r] = jnp.minimum(d, rmin[r])
                ridx[r] = jnp.where(upd, i, ridx[r])
                rI[r] = jnp.where(upd, iv, rI[r])
        return tuple(rmin), tuple(ridx), tuple(rI)

    init = (
        tuple(jnp.full((S, LN), jnp.inf, jnp.float32) for _ in range(_NSTREAM)),
        tuple(jnp.zeros((S, LN), jnp.int32) for _ in range(_NSTREAM)),
        tuple(jnp.zeros((S, LN), jnp.float32) for _ in range(_NSTREAM)),
    )
    rmin, ridx, rI = lax.fori_loop(0, n_pred // per_iter, step, init)

    bd, bi, bI = rmin[0], ridx[0], rI[0]
    for r in range(1, _NSTREAM):
        better = (rmin[r] < bd) | ((rmin[r] == bd) & (ridx[r] < bi))
        bd = jnp.where(better, rmin[r], bd)
        bi = jnp.where(better, ridx[r], bi)
        bI = jnp.where(better, rI[r], bI)
    outI_ref[0] = bI
    outidx_ref[0] = bi


def _tc_nn(pred_mz, pred_I, tgt_mz, nb):
    B, Np = pred_mz.shape
    _, Nt = tgt_mz.shape
    S = 8
    LN = Nt // S
    tgt3 = tgt_mz.reshape(B, S, LN)
    pred3 = pred_mz.reshape(B, 1, Np)
    predI3 = pred_I.reshape(B, 1, Np)
    matched_I3, matched_idx3 = pl.pallas_call(
        _nn_body,
        grid=(nb,),
        in_specs=[
            pl.BlockSpec((1, 1, Np), lambda b: (b, 0, 0), memory_space=pltpu.SMEM),
            pl.BlockSpec((1, 1, Np), lambda b: (b, 0, 0), memory_space=pltpu.SMEM),
            pl.BlockSpec((1, S, LN), lambda b: (b, 0, 0)),
        ],
        out_specs=(
            pl.BlockSpec((1, S, LN), lambda b: (b, 0, 0)),
            pl.BlockSpec((1, S, LN), lambda b: (b, 0, 0)),
        ),
        out_shape=(
            jax.ShapeDtypeStruct((nb, S, LN), jnp.float32),
            jax.ShapeDtypeStruct((nb, S, LN), jnp.int32),
        ),
    )(pred3, predI3, tgt3)
    return matched_I3.reshape(nb, Nt), matched_idx3.reshape(nb, Nt)


def _sc_nn(pred_mz, pred_I, tgt_mz, b0):
    B, Nt = tgt_mz.shape
    _, Np = pred_mz.shape
    Bs = B - b0
    info = plsc.get_sparse_core_info()
    nc, ns = info.num_cores, info.num_subcores
    nw = nc * ns
    chunk = (Bs * Nt) // nw
    per_b = Nt // chunk
    n_tv = chunk // _L
    mesh = plsc.VectorSubcoreMesh(core_axis_name="c", subcore_axis_name="s")

    @functools.partial(
        pl.kernel,
        mesh=mesh,
        out_type=(
            jax.ShapeDtypeStruct((Bs, Nt), jnp.float32),
            jax.ShapeDtypeStruct((Bs, Nt), jnp.int32),
        ),
        scratch_types=[
            pltpu.VMEM((Np,), jnp.float32),
            pltpu.VMEM((Np,), jnp.float32),
            pltpu.VMEM((chunk,), jnp.float32),
            pltpu.VMEM((chunk,), jnp.float32),
            pltpu.VMEM((chunk,), jnp.int32),
        ],
    )
    def body(pred_hbm, predI_hbm, tgt_hbm, outI_hbm, outidx_hbm,
             tab_v, tabI_v, tgt_v, oI_v, oidx_v):
        wid = lax.axis_index("s") * nc + lax.axis_index("c")
        bo = wid // per_b
        b = bo + b0
        h = wid % per_b
        pltpu.sync_copy(pred_hbm.at[b], tab_v)
        pltpu.sync_copy(predI_hbm.at[b], tabI_v)
        pltpu.sync_copy(tgt_hbm.at[b, pl.ds(h * chunk, chunk)], tgt_v)

        for g in range(n_tv // 8):
            Ts = [tgt_v[pl.ds((g * 8 + q) * _L, _L)] for q in range(8)]

            def step(k, carry):
                dmin, idx, ival = carry
                dmin, idx, ival = list(dmin), list(idx), list(ival)
                base = k * _L
                p16 = tab_v[pl.ds(base, _L)]
                i16 = tabI_v[pl.ds(base, _L)]
                for u in range(_L):
                    p = p16[u]
                    pI = i16[u]
                    i = base + u
                    for q in range(8):
                        d = jnp.abs(Ts[q] - p)
                        upd = d < dmin[q]
                        dmin[q] = jnp.minimum(d, dmin[q])
                        idx[q] = jnp.where(upd, i, idx[q])
                        ival[q] = jnp.where(upd, pI, ival[q])
                return tuple(dmin), tuple(idx), tuple(ival)

            init = (
                tuple(jnp.full((_L,), jnp.inf, jnp.float32) for _ in range(8)),
                tuple(jnp.zeros((_L,), jnp.int32) for _ in range(8)),
                tuple(jnp.zeros((_L,), jnp.float32) for _ in range(8)),
            )
            dmin, idx, ival = lax.fori_loop(0, Np // _L, step, init)
            for q in range(8):
                oidx_v[pl.ds((g * 8 + q) * _L, _L)] = idx[q]
                oI_v[pl.ds((g * 8 + q) * _L, _L)] = ival[q]

        pltpu.sync_copy(oI_v, outI_hbm.at[bo, pl.ds(h * chunk, chunk)])
        pltpu.sync_copy(oidx_v, outidx_hbm.at[bo, pl.ds(h * chunk, chunk)])

    return body(pred_mz, pred_I, tgt_mz)


@jax.jit
def kernel(pred_mz, pred_I, tgt_mz):
    bt = _TC_BATCHES
    sc_I, sc_idx = _sc_nn(pred_mz, pred_I, tgt_mz, bt)
    tc_I, tc_idx = _tc_nn(pred_mz, pred_I, tgt_mz, bt)
    matched_I = jnp.concatenate([tc_I, sc_I], axis=0)
    matched_idx = jnp.concatenate([tc_idx, sc_idx], axis=0)
    return matched_I, matched_idx
